# baseline (speedup 1.0000x reference)
import numpy as np
import jax
import jax.numpy as jnp
from jax.experimental import pallas as pl

B = 1024
SEQ = 10
HW = 16
NTAB = 19
NSZ = 7
C = 24
WC = HW * C
TC48 = SEQ * 48

BS = 32
F = BS * SEQ
R = F * HW

BS2 = 256
F2 = BS2 * SEQ

_INV_SQRT2 = float(1.0 / np.sqrt(2.0))

_SEL = np.zeros((HW, 3, HW), np.float32)
for _wi in range(HW):
    for _t in range(3):
        _w = _wi - _t + 1
        if 0 <= _w < HW:
            _SEL[_wi, _t, _w] = 1.0

_DSEL = {}
for _d in (1, 2, 4):
    m = np.zeros((SEQ, SEQ, 3), np.float32)
    for _t in range(SEQ):
        for _tap in range(3):
            _ti = _t + (_tap - 1) * _d
            if 0 <= _ti < SEQ:
                m[_ti, _t, _tap] = 1.0
    _DSEL[_d] = m

_EXP = np.kron(np.eye(SEQ, dtype=np.float32), np.ones((1, 48), np.float32))
_TSUM = np.tile(np.eye(48, dtype=np.float32), (SEQ, 1))


def _dot(a, b):
    return jax.lax.dot_general(a, b, (((1,), (0,)), ((), ())),
                               preferred_element_type=jnp.float32)


def _gelu(v):
    return v * 0.5 * (1.0 + jax.lax.erf(v * _INV_SQRT2))


def _shift_rows(a, off):
    n, c = a.shape
    if off > 0:
        return jnp.concatenate(
            [a[off:], jnp.zeros((off, c), a.dtype)], axis=0)
    if off < 0:
        return jnp.concatenate(
            [jnp.zeros((-off, c), a.dtype), a[:off]], axis=0)
    return a


def _softmax(x):
    m = jnp.max(x, axis=1, keepdims=True)
    e = jnp.exp(x - m)
    return e / jnp.sum(e, axis=1, keepdims=True)


def _conv_stack(a, m_neg, m_pos, ones8):
    return jnp.concatenate(
        [m_neg * _shift_rows(a, -1), a, m_pos * _shift_rows(a, 1), ones8],
        axis=1)


def _encoder(x2b, m_neg, m_pos, ones8, a1, a2, q, fcb):
    r1 = jax.nn.relu(_dot(_conv_stack(x2b, m_neg, m_pos, ones8), a1))
    r1b = r1.astype(jnp.bfloat16)
    r2 = jax.nn.relu(_dot(_conv_stack(r1b, m_neg, m_pos, ones8), a2))
    t = _dot(r2.astype(jnp.bfloat16), q)
    return t.reshape(F, HW, C).sum(axis=1) + fcb


def _enc_body(*refs):
    raw_ref, norm_ref = refs[0], refs[1]
    a1s, a2s, qs, fcbs = [r[...] for r in refs[2:6]]
    feat_out, amp_out = refs[6], refs[7]

    raw = raw_ref[...]
    norm = norm_ref[...]
    delta = jnp.concatenate(
        [jnp.zeros_like(norm[:, :1]), norm[:, 1:] - norm[:, :-1]], axis=1)

    hrow = jax.lax.broadcasted_iota(jnp.int32, (R, 1), 0) % HW
    m_neg = (hrow >= 1).astype(jnp.bfloat16)
    m_pos = (hrow <= HW - 2).astype(jnp.bfloat16)
    ones8 = jnp.ones((R, 8), jnp.bfloat16)

    outs = []
    for k, x4 in enumerate((raw, norm, delta)):
        x2b = x4.reshape(R, HW).astype(jnp.bfloat16)
        outs.append(_encoder(x2b, m_neg, m_pos, ones8,
                             a1s[k], a2s[k], qs[k], fcbs[k]))
    feat_out[...] = jnp.concatenate(outs, axis=1)
    amp_out[...] = raw.sum(axis=3).sum(axis=2) * np.float32(1.0 / (HW * HW))


def _seq_body(refs, n_params):
    feat_ref, amp_ref, tab_ref = refs[0], refs[1], refs[2]
    c = [r[...] for r in refs[3:3 + n_params]]
    ol, oo, org, oprob = refs[3 + n_params:]

    (ti_w, bn_sc, bn_b, wacc, bacc, gp_m, exp_m, tsum_m,
     tr_w, tb1_w, tb2_w, t2t_w, fus_w,
     sc1_w, sc2_w, so1_w, so2_w, sr1_w, sr2_w, biases) = c

    def bia(name):
        o = _BIAS_OFF[name]
        w = dict(_BIAS_PARTS)[name]
        return biases[:, o:o + w]

    gp_b, tr_b, ln_g, ln_b = bia('gp_b'), bia('tr_b'), bia('ln_g'), bia('ln_b')
    tb1_b, tb2_b, t2t_b = bia('tb1_b'), bia('tb2_b'), bia('t2t_b')
    fus_b, sc1_b, sc2_b = bia('fus_b'), bia('sc1_b'), bia('sc2_b')
    so1_b, so2_b, sr1_b, sr2_b = (bia('so1_b'), bia('so2_b'),
                                  bia('sr1_b'), bia('sr2_b'))

    feat = feat_ref[...]
    amp = amp_ref[...]
    tab = tab_ref[...]

    s = jax.nn.relu(_dot(feat, ti_w) * bn_sc + bn_b)
    s3 = s.reshape(BS2, SEQ, 48)
    st = jnp.concatenate([s3[:, t] for t in range(SEQ)], axis=1)

    for i in range(3):
        st = jax.nn.relu(st + _dot(st, wacc[i]) + bacc[i])

    scores = _dot(st, gp_m) + gp_b
    attn = _softmax(scores)

    thr = amp.mean(axis=1, keepdims=True)
    m_hi = (amp >= thr).astype(jnp.float32)
    m_lo = 1.0 - m_hi

    gfeat = _dot(st * _dot(attn, exp_m), tsum_m)
    ph = (_dot(st * _dot(m_hi, exp_m), tsum_m) /
          (jnp.sum(m_hi, axis=1, keepdims=True) + 1e-6))
    plo = (_dot(st * _dot(m_lo, exp_m), tsum_m) /
           (jnp.sum(m_lo, axis=1, keepdims=True) + 1e-6))

    fused = jnp.concatenate([gfeat, ph, plo], axis=1)
    trunk = jax.nn.relu(_dot(fused, tr_w) + tr_b)

    mu = tab.mean(axis=1, keepdims=True)
    xc = tab - mu
    var = (xc * xc).mean(axis=1, keepdims=True)
    tn = xc / jnp.sqrt(var + 1e-5) * ln_g + ln_b
    tf = _gelu(_dot(tn, tb1_w) + tb1_b)
    tf = _gelu(_dot(tf, tb2_w) + tb2_b)
    tp = _dot(tf, t2t_w) + t2t_b

    hyb = jax.nn.relu(
        _dot(jnp.concatenate([trunk, tp, trunk * tp], axis=1), fus_w) + fus_b)
    sl = _dot(jax.nn.relu(_dot(hyb, sc1_w) + sc1_b), sc2_w) + sc2_b
    so = _dot(jax.nn.relu(_dot(hyb, so1_w) + so1_b), so2_w) + so2_b
    sp = _softmax(sl)
    vals = (jax.lax.broadcasted_iota(jnp.int32, (1, NSZ), 1).astype(
        jnp.float32) * np.float32(1.0 / (NSZ - 1)))
    expected = jnp.sum(sp * vals, axis=1, keepdims=True)
    res = 0.35 * jnp.tanh(
        _dot(jax.nn.relu(_dot(hyb, sr1_w) + sr1_b), sr2_w) + sr2_b)
    rg = jnp.clip(expected + res, 0.0, 1.0)

    ol[...] = sl
    oo[...] = so
    org[...] = rg
    oprob[...] = sp


def _prep_enc_params(p):
    sel = jnp.asarray(_SEL)
    encs = ('amp', 'shp', 'dlt')
    w1 = jnp.stack([p[e + '_c1w'].reshape(3, 3, C) for e in encs])
    b1 = jnp.stack([p[e + '_c1b'] for e in encs])
    w2 = jnp.stack([p[e + '_c2w'] for e in encs])
    b2 = jnp.stack([p[e + '_c2b'] for e in encs])
    fcw = jnp.stack([p[e + '_fcw'] for e in encs])
    fcb = jnp.stack([p[e + '_fcb'] for e in encs])

    a1 = jnp.einsum('itw,eotc->eoiwc', sel, w1).reshape(3, 3 * HW, WC)
    b1t = jnp.tile(b1, (1, HW)).reshape(3, 1, WC)
    a1 = jnp.concatenate(
        [a1, b1t, jnp.zeros((3, 7, WC), jnp.float32)], axis=1)
    a2 = jnp.einsum('itw,eotcd->eoicwd', sel, w2).reshape(3, 3 * WC, WC)
    b2t = jnp.tile(b2, (1, HW)).reshape(3, 1, WC)
    a2 = jnp.concatenate(
        [a2, b2t, jnp.zeros((3, 7, WC), jnp.float32)], axis=1)
    q = jnp.tile(fcw * np.float32(1.0 / (HW * HW)), (1, HW, 1))
    return [a1.astype(jnp.bfloat16), a2.astype(jnp.bfloat16),
            q.astype(jnp.bfloat16), fcb.reshape(3, 1, C)]


_BIAS_PARTS = [('gp_b', 1), ('tr_b', 64), ('ln_g', NTAB), ('ln_b', NTAB),
               ('tb1_b', 64), ('tb2_b', 64), ('t2t_b', 64), ('fus_b', 96),
               ('sc1_b', 96), ('sc2_b', NSZ), ('so1_b', 48), ('so2_b', 6),
               ('sr1_b', 96), ('sr2_b', 1)]
_BIAS_OFF = {}
_off = 0
for _n, _w in _BIAS_PARTS:
    _BIAS_OFF[_n] = _off
    _off += _w
_BIAS_TOT = _off


def _prep_seq_params(p):
    dsel = jnp.asarray(np.stack([_DSEL[d] for d in (1, 2, 4)]))
    w_all = jnp.stack([
        jnp.stack([p[f'blk{i}_d{d}_w'] for d in (1, 2, 4)])
        for i in range(3)])
    b_all = jnp.stack([
        jnp.stack([p[f'blk{i}_d{d}_b'] for d in (1, 2, 4)])
        for i in range(3)])
    wacc = jnp.einsum('dutp,idpcm->iuctm', dsel,
                      w_all).reshape(3, TC48, TC48)
    bacc = jnp.tile(b_all.sum(axis=1), (1, SEQ)).reshape(3, 1, TC48)
    gp_m = jnp.kron(jnp.eye(SEQ, dtype=jnp.float32),
                    p['gp_w'].reshape(48, 1))
    biases = jnp.concatenate(
        [p[n].reshape(-1) for n, _ in _BIAS_PARTS]).reshape(1, _BIAS_TOT)
    out = [p['ti_w'].reshape(72, 48),
           p['bn_g'].reshape(1, 48) * np.float32(1.0 / np.sqrt(1.0 + 1e-5)),
           p['bn_b'].reshape(1, 48),
           wacc, bacc, gp_m,
           jnp.asarray(_EXP), jnp.asarray(_TSUM),
           p['tr_w'], p['tb1_w'], p['tb2_w'], p['t2t_w'], p['fus_w'],
           p['sc1_w'], p['sc2_w'], p['so1_w'], p['so2_w'], p['sr1_w'],
           p['sr2_w'], biases]
    return out


def _full_spec(a):
    nd = a.ndim
    return pl.BlockSpec(a.shape, lambda i, _nd=nd: (0,) * _nd)


def kernel(raw_window, norm_window, tabular_x, params):
    enc_consts = _prep_enc_params(params)
    seq_consts = _prep_seq_params(params)

    enc_fn = pl.pallas_call(
        _enc_body,
        grid=(B // BS,),
        in_specs=[
            pl.BlockSpec((BS, SEQ, HW, HW), lambda i: (i, 0, 0, 0)),
            pl.BlockSpec((BS, SEQ, HW, HW), lambda i: (i, 0, 0, 0)),
        ] + [_full_spec(a) for a in enc_consts],
        out_specs=[
            pl.BlockSpec((F, 72), lambda i: (i, 0)),
            pl.BlockSpec((BS, SEQ), lambda i: (i, 0)),
        ],
        out_shape=[
            jax.ShapeDtypeStruct((B * SEQ, 72), jnp.float32),
            jax.ShapeDtypeStruct((B, SEQ), jnp.float32),
        ],
    )
    feat, amp = enc_fn(raw_window, norm_window, *enc_consts)

    n_seq = len(seq_consts)
    seq_fn = pl.pallas_call(
        lambda *refs: _seq_body(refs, n_seq),
        grid=(B // BS2,),
        in_specs=[
            pl.BlockSpec((F2, 72), lambda i: (i, 0)),
            pl.BlockSpec((BS2, SEQ), lambda i: (i, 0)),
            pl.BlockSpec((BS2, NTAB), lambda i: (i, 0)),
        ] + [_full_spec(a) for a in seq_consts],
        out_specs=[
            pl.BlockSpec((BS2, NSZ), lambda i: (i, 0)),
            pl.BlockSpec((BS2, 6), lambda i: (i, 0)),
            pl.BlockSpec((BS2, 1), lambda i: (i, 0)),
            pl.BlockSpec((BS2, NSZ), lambda i: (i, 0)),
        ],
        out_shape=[
            jax.ShapeDtypeStruct((B, NSZ), jnp.float32),
            jax.ShapeDtypeStruct((B, 6), jnp.float32),
            jax.ShapeDtypeStruct((B, 1), jnp.float32),
            jax.ShapeDtypeStruct((B, NSZ), jnp.float32),
        ],
    )
    return tuple(seq_fn(feat, amp, tabular_x, *seq_consts))


# speedup vs baseline: 1.0167x; 1.0167x over previous
import numpy as np
import jax
import jax.numpy as jnp
from jax.experimental import pallas as pl

B = 1024
SEQ = 10
HW = 16
NTAB = 19
NSZ = 7
C = 24
WC = HW * C
TC48 = SEQ * 48

BS = 16
F = BS * SEQ
R = F * HW

BS2 = 256
F2 = BS2 * SEQ

_INV_SQRT2 = float(1.0 / np.sqrt(2.0))

_SEL = np.zeros((HW, 3, HW), np.float32)
for _wi in range(HW):
    for _t in range(3):
        _w = _wi - _t + 1
        if 0 <= _w < HW:
            _SEL[_wi, _t, _w] = 1.0

_DSEL = {}
for _d in (1, 2, 4):
    m = np.zeros((SEQ, SEQ, 3), np.float32)
    for _t in range(SEQ):
        for _tap in range(3):
            _ti = _t + (_tap - 1) * _d
            if 0 <= _ti < SEQ:
                m[_ti, _t, _tap] = 1.0
    _DSEL[_d] = m

_EXP = np.kron(np.eye(SEQ, dtype=np.float32), np.ones((1, 48), np.float32))
_TSUM = np.tile(np.eye(48, dtype=np.float32), (SEQ, 1))


def _dot(a, b):
    return jax.lax.dot_general(a, b, (((1,), (0,)), ((), ())),
                               preferred_element_type=jnp.float32)


def _gelu(v):
    return v * 0.5 * (1.0 + jax.lax.erf(v * _INV_SQRT2))


def _shift_rows(a, off):
    n, c = a.shape
    if off > 0:
        return jnp.concatenate(
            [a[off:], jnp.zeros((off, c), a.dtype)], axis=0)
    if off < 0:
        return jnp.concatenate(
            [jnp.zeros((-off, c), a.dtype), a[:off]], axis=0)
    return a


def _softmax(x):
    m = jnp.max(x, axis=1, keepdims=True)
    e = jnp.exp(x - m)
    return e / jnp.sum(e, axis=1, keepdims=True)


def _conv_stack(a, m_neg, m_pos, ones8):
    return jnp.concatenate(
        [m_neg * _shift_rows(a, -1), a, m_pos * _shift_rows(a, 1), ones8],
        axis=1)


def _encoder(x2b, m_neg, m_pos, ones8, a1, a2, q, fcb):
    r1 = jax.nn.relu(_dot(_conv_stack(x2b, m_neg, m_pos, ones8), a1))
    r1b = r1.astype(jnp.bfloat16)
    r2 = jax.nn.relu(_dot(_conv_stack(r1b, m_neg, m_pos, ones8), a2))
    t = _dot(r2.astype(jnp.bfloat16), q)
    return t.reshape(F, HW, C).sum(axis=1) + fcb


def _enc_body(*refs):
    raw_ref, norm_ref = refs[0], refs[1]
    a1s, a2s, qs, fcbs = [r[...] for r in refs[2:6]]
    feat_out, amp_out = refs[6], refs[7]

    raw = raw_ref[...]
    norm = norm_ref[...]
    delta = jnp.concatenate(
        [jnp.zeros_like(norm[:, :1]), norm[:, 1:] - norm[:, :-1]], axis=1)

    hrow = jax.lax.broadcasted_iota(jnp.int32, (R, 1), 0) % HW
    m_neg = (hrow >= 1).astype(jnp.bfloat16)
    m_pos = (hrow <= HW - 2).astype(jnp.bfloat16)
    ones8 = jnp.ones((R, 8), jnp.bfloat16)

    outs = []
    for k, x4 in enumerate((raw, norm, delta)):
        x2b = x4.reshape(R, HW).astype(jnp.bfloat16)
        outs.append(_encoder(x2b, m_neg, m_pos, ones8,
                             a1s[k], a2s[k], qs[k], fcbs[k]))
    feat_out[...] = jnp.concatenate(outs, axis=1)
    amp_out[...] = raw.sum(axis=3).sum(axis=2) * np.float32(1.0 / (HW * HW))


def _seq_body(refs, n_params):
    feat_ref, amp_ref, tab_ref = refs[0], refs[1], refs[2]
    c = [r[...] for r in refs[3:3 + n_params]]
    ol, oo, org, oprob = refs[3 + n_params:]

    (ti_w, bn_sc, bn_b, wacc, bacc, gp_m, exp_m, tsum_m,
     tr_w, tb1_w, tb2_w, t2t_w, fus_w,
     sc1_w, sc2_w, so1_w, so2_w, sr1_w, sr2_w, biases) = c

    def bia(name):
        o = _BIAS_OFF[name]
        w = dict(_BIAS_PARTS)[name]
        return biases[:, o:o + w]

    gp_b, tr_b, ln_g, ln_b = bia('gp_b'), bia('tr_b'), bia('ln_g'), bia('ln_b')
    tb1_b, tb2_b, t2t_b = bia('tb1_b'), bia('tb2_b'), bia('t2t_b')
    fus_b, sc1_b, sc2_b = bia('fus_b'), bia('sc1_b'), bia('sc2_b')
    so1_b, so2_b, sr1_b, sr2_b = (bia('so1_b'), bia('so2_b'),
                                  bia('sr1_b'), bia('sr2_b'))

    feat = feat_ref[...]
    amp = amp_ref[...]
    tab = tab_ref[...]

    s = jax.nn.relu(_dot(feat, ti_w) * bn_sc + bn_b)
    s3 = s.reshape(BS2, SEQ, 48)
    st = jnp.concatenate([s3[:, t] for t in range(SEQ)], axis=1)

    for i in range(3):
        st = jax.nn.relu(st + _dot(st, wacc[i]) + bacc[i])

    scores = _dot(st, gp_m) + gp_b
    attn = _softmax(scores)

    thr = amp.mean(axis=1, keepdims=True)
    m_hi = (amp >= thr).astype(jnp.float32)
    m_lo = 1.0 - m_hi

    gfeat = _dot(st * _dot(attn, exp_m), tsum_m)
    ph = (_dot(st * _dot(m_hi, exp_m), tsum_m) /
          (jnp.sum(m_hi, axis=1, keepdims=True) + 1e-6))
    plo = (_dot(st * _dot(m_lo, exp_m), tsum_m) /
           (jnp.sum(m_lo, axis=1, keepdims=True) + 1e-6))

    fused = jnp.concatenate([gfeat, ph, plo], axis=1)
    trunk = jax.nn.relu(_dot(fused, tr_w) + tr_b)

    mu = tab.mean(axis=1, keepdims=True)
    xc = tab - mu
    var = (xc * xc).mean(axis=1, keepdims=True)
    tn = xc / jnp.sqrt(var + 1e-5) * ln_g + ln_b
    tf = _gelu(_dot(tn, tb1_w) + tb1_b)
    tf = _gelu(_dot(tf, tb2_w) + tb2_b)
    tp = _dot(tf, t2t_w) + t2t_b

    hyb = jax.nn.relu(
        _dot(jnp.concatenate([trunk, tp, trunk * tp], axis=1), fus_w) + fus_b)
    sl = _dot(jax.nn.relu(_dot(hyb, sc1_w) + sc1_b), sc2_w) + sc2_b
    so = _dot(jax.nn.relu(_dot(hyb, so1_w) + so1_b), so2_w) + so2_b
    sp = _softmax(sl)
    vals = (jax.lax.broadcasted_iota(jnp.int32, (1, NSZ), 1).astype(
        jnp.float32) * np.float32(1.0 / (NSZ - 1)))
    expected = jnp.sum(sp * vals, axis=1, keepdims=True)
    res = 0.35 * jnp.tanh(
        _dot(jax.nn.relu(_dot(hyb, sr1_w) + sr1_b), sr2_w) + sr2_b)
    rg = jnp.clip(expected + res, 0.0, 1.0)

    ol[...] = sl
    oo[...] = so
    org[...] = rg
    oprob[...] = sp


def _prep_enc_params(p):
    sel = jnp.asarray(_SEL)
    encs = ('amp', 'shp', 'dlt')
    w1 = jnp.stack([p[e + '_c1w'].reshape(3, 3, C) for e in encs])
    b1 = jnp.stack([p[e + '_c1b'] for e in encs])
    w2 = jnp.stack([p[e + '_c2w'] for e in encs])
    b2 = jnp.stack([p[e + '_c2b'] for e in encs])
    fcw = jnp.stack([p[e + '_fcw'] for e in encs])
    fcb = jnp.stack([p[e + '_fcb'] for e in encs])

    a1 = jnp.einsum('itw,eotc->eoiwc', sel, w1).reshape(3, 3 * HW, WC)
    b1t = jnp.tile(b1, (1, HW)).reshape(3, 1, WC)
    a1 = jnp.concatenate(
        [a1, b1t, jnp.zeros((3, 7, WC), jnp.float32)], axis=1)
    a2 = jnp.einsum('itw,eotcd->eoicwd', sel, w2).reshape(3, 3 * WC, WC)
    b2t = jnp.tile(b2, (1, HW)).reshape(3, 1, WC)
    a2 = jnp.concatenate(
        [a2, b2t, jnp.zeros((3, 7, WC), jnp.float32)], axis=1)
    q = jnp.tile(fcw * np.float32(1.0 / (HW * HW)), (1, HW, 1))
    return [a1.astype(jnp.bfloat16), a2.astype(jnp.bfloat16),
            q.astype(jnp.bfloat16), fcb.reshape(3, 1, C)]


_BIAS_PARTS = [('gp_b', 1), ('tr_b', 64), ('ln_g', NTAB), ('ln_b', NTAB),
               ('tb1_b', 64), ('tb2_b', 64), ('t2t_b', 64), ('fus_b', 96),
               ('sc1_b', 96), ('sc2_b', NSZ), ('so1_b', 48), ('so2_b', 6),
               ('sr1_b', 96), ('sr2_b', 1)]
_BIAS_OFF = {}
_off = 0
for _n, _w in _BIAS_PARTS:
    _BIAS_OFF[_n] = _off
    _off += _w
_BIAS_TOT = _off


def _prep_seq_params(p):
    dsel = jnp.asarray(np.stack([_DSEL[d] for d in (1, 2, 4)]))
    w_all = jnp.stack([
        jnp.stack([p[f'blk{i}_d{d}_w'] for d in (1, 2, 4)])
        for i in range(3)])
    b_all = jnp.stack([
        jnp.stack([p[f'blk{i}_d{d}_b'] for d in (1, 2, 4)])
        for i in range(3)])
    wacc = jnp.einsum('dutp,idpcm->iuctm', dsel,
                      w_all).reshape(3, TC48, TC48)
    bacc = jnp.tile(b_all.sum(axis=1), (1, SEQ)).reshape(3, 1, TC48)
    gp_m = jnp.kron(jnp.eye(SEQ, dtype=jnp.float32),
                    p['gp_w'].reshape(48, 1))
    biases = jnp.concatenate(
        [p[n].reshape(-1) for n, _ in _BIAS_PARTS]).reshape(1, _BIAS_TOT)
    out = [p['ti_w'].reshape(72, 48),
           p['bn_g'].reshape(1, 48) * np.float32(1.0 / np.sqrt(1.0 + 1e-5)),
           p['bn_b'].reshape(1, 48),
           wacc, bacc, gp_m,
           jnp.asarray(_EXP), jnp.asarray(_TSUM),
           p['tr_w'], p['tb1_w'], p['tb2_w'], p['t2t_w'], p['fus_w'],
           p['sc1_w'], p['sc2_w'], p['so1_w'], p['so2_w'], p['sr1_w'],
           p['sr2_w'], biases]
    return out


def _full_spec(a):
    nd = a.ndim
    return pl.BlockSpec(a.shape, lambda i, _nd=nd: (0,) * _nd)


def kernel(raw_window, norm_window, tabular_x, params):
    enc_consts = _prep_enc_params(params)
    seq_consts = _prep_seq_params(params)

    enc_fn = pl.pallas_call(
        _enc_body,
        grid=(B // BS,),
        in_specs=[
            pl.BlockSpec((BS, SEQ, HW, HW), lambda i: (i, 0, 0, 0)),
            pl.BlockSpec((BS, SEQ, HW, HW), lambda i: (i, 0, 0, 0)),
        ] + [_full_spec(a) for a in enc_consts],
        out_specs=[
            pl.BlockSpec((F, 72), lambda i: (i, 0)),
            pl.BlockSpec((BS, SEQ), lambda i: (i, 0)),
        ],
        out_shape=[
            jax.ShapeDtypeStruct((B * SEQ, 72), jnp.float32),
            jax.ShapeDtypeStruct((B, SEQ), jnp.float32),
        ],
    )
    feat, amp = enc_fn(raw_window, norm_window, *enc_consts)

    n_seq = len(seq_consts)
    seq_fn = pl.pallas_call(
        lambda *refs: _seq_body(refs, n_seq),
        grid=(B // BS2,),
        in_specs=[
            pl.BlockSpec((F2, 72), lambda i: (i, 0)),
            pl.BlockSpec((BS2, SEQ), lambda i: (i, 0)),
            pl.BlockSpec((BS2, NTAB), lambda i: (i, 0)),
        ] + [_full_spec(a) for a in seq_consts],
        out_specs=[
            pl.BlockSpec((BS2, NSZ), lambda i: (i, 0)),
            pl.BlockSpec((BS2, 6), lambda i: (i, 0)),
            pl.BlockSpec((BS2, 1), lambda i: (i, 0)),
            pl.BlockSpec((BS2, NSZ), lambda i: (i, 0)),
        ],
        out_shape=[
            jax.ShapeDtypeStruct((B, NSZ), jnp.float32),
            jax.ShapeDtypeStruct((B, 6), jnp.float32),
            jax.ShapeDtypeStruct((B, 1), jnp.float32),
            jax.ShapeDtypeStruct((B, NSZ), jnp.float32),
        ],
    )
    return tuple(seq_fn(feat, amp, tabular_x, *seq_consts))


# speedup vs baseline: 1.0220x; 1.0052x over previous
import numpy as np
import jax
import jax.numpy as jnp
from jax.experimental import pallas as pl

B = 1024
SEQ = 10
HW = 16
NTAB = 19
NSZ = 7
C = 24
WC = HW * C
TC48 = SEQ * 48

BS = 8
F = BS * SEQ
R = F * HW

BS2 = 256
F2 = BS2 * SEQ

_INV_SQRT2 = float(1.0 / np.sqrt(2.0))

_SEL = np.zeros((HW, 3, HW), np.float32)
for _wi in range(HW):
    for _t in range(3):
        _w = _wi - _t + 1
        if 0 <= _w < HW:
            _SEL[_wi, _t, _w] = 1.0

_DSEL = {}
for _d in (1, 2, 4):
    m = np.zeros((SEQ, SEQ, 3), np.float32)
    for _t in range(SEQ):
        for _tap in range(3):
            _ti = _t + (_tap - 1) * _d
            if 0 <= _ti < SEQ:
                m[_ti, _t, _tap] = 1.0
    _DSEL[_d] = m

_EXP = np.kron(np.eye(SEQ, dtype=np.float32), np.ones((1, 48), np.float32))
_TSUM = np.tile(np.eye(48, dtype=np.float32), (SEQ, 1))


def _dot(a, b):
    return jax.lax.dot_general(a, b, (((1,), (0,)), ((), ())),
                               preferred_element_type=jnp.float32)


def _gelu(v):
    return v * 0.5 * (1.0 + jax.lax.erf(v * _INV_SQRT2))


def _shift_rows(a, off):
    n, c = a.shape
    if off > 0:
        return jnp.concatenate(
            [a[off:], jnp.zeros((off, c), a.dtype)], axis=0)
    if off < 0:
        return jnp.concatenate(
            [jnp.zeros((-off, c), a.dtype), a[:off]], axis=0)
    return a


def _softmax(x):
    m = jnp.max(x, axis=1, keepdims=True)
    e = jnp.exp(x - m)
    return e / jnp.sum(e, axis=1, keepdims=True)


def _conv_stack(a, m_neg, m_pos, ones8):
    return jnp.concatenate(
        [m_neg * _shift_rows(a, -1), a, m_pos * _shift_rows(a, 1), ones8],
        axis=1)


def _encoder(x2b, m_neg, m_pos, ones8, a1, a2, q, fcb):
    r1 = jax.nn.relu(_dot(_conv_stack(x2b, m_neg, m_pos, ones8), a1))
    r1b = r1.astype(jnp.bfloat16)
    r2 = jax.nn.relu(_dot(_conv_stack(r1b, m_neg, m_pos, ones8), a2))
    t = _dot(r2.astype(jnp.bfloat16), q)
    return t.reshape(F, HW, C).sum(axis=1) + fcb


def _enc_body(*refs):
    raw_ref, norm_ref = refs[0], refs[1]
    a1s, a2s, qs, fcbs = [r[...] for r in refs[2:6]]
    feat_out, amp_out = refs[6], refs[7]

    raw = raw_ref[...]
    norm = norm_ref[...]
    delta = jnp.concatenate(
        [jnp.zeros_like(norm[:, :1]), norm[:, 1:] - norm[:, :-1]], axis=1)

    hrow = jax.lax.broadcasted_iota(jnp.int32, (R, 1), 0) % HW
    m_neg = (hrow >= 1).astype(jnp.bfloat16)
    m_pos = (hrow <= HW - 2).astype(jnp.bfloat16)
    ones8 = jnp.ones((R, 8), jnp.bfloat16)

    outs = []
    for k, x4 in enumerate((raw, norm, delta)):
        x2b = x4.reshape(R, HW).astype(jnp.bfloat16)
        outs.append(_encoder(x2b, m_neg, m_pos, ones8,
                             a1s[k], a2s[k], qs[k], fcbs[k]))
    feat_out[...] = jnp.concatenate(outs, axis=1)
    amp_out[...] = raw.sum(axis=3).sum(axis=2) * np.float32(1.0 / (HW * HW))


def _seq_body(refs, n_params):
    feat_ref, amp_ref, tab_ref = refs[0], refs[1], refs[2]
    c = [r[...] for r in refs[3:3 + n_params]]
    ol, oo, org, oprob = refs[3 + n_params:]

    (ti_w, bn_sc, bn_b, wacc, bacc, gp_m, exp_m, tsum_m,
     tr_w, tb1_w, tb2_w, t2t_w, fus_w,
     sc1_w, sc2_w, so1_w, so2_w, sr1_w, sr2_w, biases) = c

    def bia(name):
        o = _BIAS_OFF[name]
        w = dict(_BIAS_PARTS)[name]
        return biases[:, o:o + w]

    gp_b, tr_b, ln_g, ln_b = bia('gp_b'), bia('tr_b'), bia('ln_g'), bia('ln_b')
    tb1_b, tb2_b, t2t_b = bia('tb1_b'), bia('tb2_b'), bia('t2t_b')
    fus_b, sc1_b, sc2_b = bia('fus_b'), bia('sc1_b'), bia('sc2_b')
    so1_b, so2_b, sr1_b, sr2_b = (bia('so1_b'), bia('so2_b'),
                                  bia('sr1_b'), bia('sr2_b'))

    feat = feat_ref[...]
    amp = amp_ref[...]
    tab = tab_ref[...]

    s = jax.nn.relu(_dot(feat, ti_w) * bn_sc + bn_b)
    s3 = s.reshape(BS2, SEQ, 48)
    st = jnp.concatenate([s3[:, t] for t in range(SEQ)], axis=1)

    for i in range(3):
        st = jax.nn.relu(st + _dot(st, wacc[i]) + bacc[i])

    scores = _dot(st, gp_m) + gp_b
    attn = _softmax(scores)

    thr = amp.mean(axis=1, keepdims=True)
    m_hi = (amp >= thr).astype(jnp.float32)
    m_lo = 1.0 - m_hi

    gfeat = _dot(st * _dot(attn, exp_m), tsum_m)
    ph = (_dot(st * _dot(m_hi, exp_m), tsum_m) /
          (jnp.sum(m_hi, axis=1, keepdims=True) + 1e-6))
    plo = (_dot(st * _dot(m_lo, exp_m), tsum_m) /
           (jnp.sum(m_lo, axis=1, keepdims=True) + 1e-6))

    fused = jnp.concatenate([gfeat, ph, plo], axis=1)
    trunk = jax.nn.relu(_dot(fused, tr_w) + tr_b)

    mu = tab.mean(axis=1, keepdims=True)
    xc = tab - mu
    var = (xc * xc).mean(axis=1, keepdims=True)
    tn = xc / jnp.sqrt(var + 1e-5) * ln_g + ln_b
    tf = _gelu(_dot(tn, tb1_w) + tb1_b)
    tf = _gelu(_dot(tf, tb2_w) + tb2_b)
    tp = _dot(tf, t2t_w) + t2t_b

    hyb = jax.nn.relu(
        _dot(jnp.concatenate([trunk, tp, trunk * tp], axis=1), fus_w) + fus_b)
    sl = _dot(jax.nn.relu(_dot(hyb, sc1_w) + sc1_b), sc2_w) + sc2_b
    so = _dot(jax.nn.relu(_dot(hyb, so1_w) + so1_b), so2_w) + so2_b
    sp = _softmax(sl)
    vals = (jax.lax.broadcasted_iota(jnp.int32, (1, NSZ), 1).astype(
        jnp.float32) * np.float32(1.0 / (NSZ - 1)))
    expected = jnp.sum(sp * vals, axis=1, keepdims=True)
    res = 0.35 * jnp.tanh(
        _dot(jax.nn.relu(_dot(hyb, sr1_w) + sr1_b), sr2_w) + sr2_b)
    rg = jnp.clip(expected + res, 0.0, 1.0)

    ol[...] = sl
    oo[...] = so
    org[...] = rg
    oprob[...] = sp


def _prep_enc_params(p):
    sel = jnp.asarray(_SEL)
    encs = ('amp', 'shp', 'dlt')
    w1 = jnp.stack([p[e + '_c1w'].reshape(3, 3, C) for e in encs])
    b1 = jnp.stack([p[e + '_c1b'] for e in encs])
    w2 = jnp.stack([p[e + '_c2w'] for e in encs])
    b2 = jnp.stack([p[e + '_c2b'] for e in encs])
    fcw = jnp.stack([p[e + '_fcw'] for e in encs])
    fcb = jnp.stack([p[e + '_fcb'] for e in encs])

    a1 = jnp.einsum('itw,eotc->eoiwc', sel, w1).reshape(3, 3 * HW, WC)
    b1t = jnp.tile(b1, (1, HW)).reshape(3, 1, WC)
    a1 = jnp.concatenate(
        [a1, b1t, jnp.zeros((3, 7, WC), jnp.float32)], axis=1)
    a2 = jnp.einsum('itw,eotcd->eoicwd', sel, w2).reshape(3, 3 * WC, WC)
    b2t = jnp.tile(b2, (1, HW)).reshape(3, 1, WC)
    a2 = jnp.concatenate(
        [a2, b2t, jnp.zeros((3, 7, WC), jnp.float32)], axis=1)
    q = jnp.tile(fcw * np.float32(1.0 / (HW * HW)), (1, HW, 1))
    return [a1.astype(jnp.bfloat16), a2.astype(jnp.bfloat16),
            q.astype(jnp.bfloat16), fcb.reshape(3, 1, C)]


_BIAS_PARTS = [('gp_b', 1), ('tr_b', 64), ('ln_g', NTAB), ('ln_b', NTAB),
               ('tb1_b', 64), ('tb2_b', 64), ('t2t_b', 64), ('fus_b', 96),
               ('sc1_b', 96), ('sc2_b', NSZ), ('so1_b', 48), ('so2_b', 6),
               ('sr1_b', 96), ('sr2_b', 1)]
_BIAS_OFF = {}
_off = 0
for _n, _w in _BIAS_PARTS:
    _BIAS_OFF[_n] = _off
    _off += _w
_BIAS_TOT = _off


def _prep_seq_params(p):
    dsel = jnp.asarray(np.stack([_DSEL[d] for d in (1, 2, 4)]))
    w_all = jnp.stack([
        jnp.stack([p[f'blk{i}_d{d}_w'] for d in (1, 2, 4)])
        for i in range(3)])
    b_all = jnp.stack([
        jnp.stack([p[f'blk{i}_d{d}_b'] for d in (1, 2, 4)])
        for i in range(3)])
    wacc = jnp.einsum('dutp,idpcm->iuctm', dsel,
                      w_all).reshape(3, TC48, TC48)
    bacc = jnp.tile(b_all.sum(axis=1), (1, SEQ)).reshape(3, 1, TC48)
    gp_m = jnp.kron(jnp.eye(SEQ, dtype=jnp.float32),
                    p['gp_w'].reshape(48, 1))
    biases = jnp.concatenate(
        [p[n].reshape(-1) for n, _ in _BIAS_PARTS]).reshape(1, _BIAS_TOT)
    out = [p['ti_w'].reshape(72, 48),
           p['bn_g'].reshape(1, 48) * np.float32(1.0 / np.sqrt(1.0 + 1e-5)),
           p['bn_b'].reshape(1, 48),
           wacc, bacc, gp_m,
           jnp.asarray(_EXP), jnp.asarray(_TSUM),
           p['tr_w'], p['tb1_w'], p['tb2_w'], p['t2t_w'], p['fus_w'],
           p['sc1_w'], p['sc2_w'], p['so1_w'], p['so2_w'], p['sr1_w'],
           p['sr2_w'], biases]
    return out


def _full_spec(a):
    nd = a.ndim
    return pl.BlockSpec(a.shape, lambda i, _nd=nd: (0,) * _nd)


def kernel(raw_window, norm_window, tabular_x, params):
    enc_consts = _prep_enc_params(params)
    seq_consts = _prep_seq_params(params)

    enc_fn = pl.pallas_call(
        _enc_body,
        grid=(B // BS,),
        in_specs=[
            pl.BlockSpec((BS, SEQ, HW, HW), lambda i: (i, 0, 0, 0)),
            pl.BlockSpec((BS, SEQ, HW, HW), lambda i: (i, 0, 0, 0)),
        ] + [_full_spec(a) for a in enc_consts],
        out_specs=[
            pl.BlockSpec((F, 72), lambda i: (i, 0)),
            pl.BlockSpec((BS, SEQ), lambda i: (i, 0)),
        ],
        out_shape=[
            jax.ShapeDtypeStruct((B * SEQ, 72), jnp.float32),
            jax.ShapeDtypeStruct((B, SEQ), jnp.float32),
        ],
    )
    feat, amp = enc_fn(raw_window, norm_window, *enc_consts)

    n_seq = len(seq_consts)
    seq_fn = pl.pallas_call(
        lambda *refs: _seq_body(refs, n_seq),
        grid=(B // BS2,),
        in_specs=[
            pl.BlockSpec((F2, 72), lambda i: (i, 0)),
            pl.BlockSpec((BS2, SEQ), lambda i: (i, 0)),
            pl.BlockSpec((BS2, NTAB), lambda i: (i, 0)),
        ] + [_full_spec(a) for a in seq_consts],
        out_specs=[
            pl.BlockSpec((BS2, NSZ), lambda i: (i, 0)),
            pl.BlockSpec((BS2, 6), lambda i: (i, 0)),
            pl.BlockSpec((BS2, 1), lambda i: (i, 0)),
            pl.BlockSpec((BS2, NSZ), lambda i: (i, 0)),
        ],
        out_shape=[
            jax.ShapeDtypeStruct((B, NSZ), jnp.float32),
            jax.ShapeDtypeStruct((B, 6), jnp.float32),
            jax.ShapeDtypeStruct((B, 1), jnp.float32),
            jax.ShapeDtypeStruct((B, NSZ), jnp.float32),
        ],
    )
    return tuple(seq_fn(feat, amp, tabular_x, *seq_consts))


# speedup vs baseline: 1.0313x; 1.0091x over previous
import numpy as np
import jax
import jax.numpy as jnp
from jax.experimental import pallas as pl

B = 1024
SEQ = 10
HW = 16
NTAB = 19
NSZ = 7
C = 24
WC = HW * C
TC48 = SEQ * 48

BS = 8
F = BS * SEQ
R = F * HW

BS2 = 256
F2 = BS2 * SEQ

_INV_SQRT2 = float(1.0 / np.sqrt(2.0))

_SEL = np.zeros((HW, 3, HW), np.float32)
for _wi in range(HW):
    for _t in range(3):
        _w = _wi - _t + 1
        if 0 <= _w < HW:
            _SEL[_wi, _t, _w] = 1.0

_DSEL = {}
for _d in (1, 2, 4):
    m = np.zeros((SEQ, SEQ, 3), np.float32)
    for _t in range(SEQ):
        for _tap in range(3):
            _ti = _t + (_tap - 1) * _d
            if 0 <= _ti < SEQ:
                m[_ti, _t, _tap] = 1.0
    _DSEL[_d] = m

_EXP = np.kron(np.eye(SEQ, dtype=np.float32), np.ones((1, 48), np.float32))
_TSUM = np.tile(np.eye(48, dtype=np.float32), (SEQ, 1))


def _dot(a, b):
    return jax.lax.dot_general(a, b, (((1,), (0,)), ((), ())),
                               preferred_element_type=jnp.float32)


def _gelu(v):
    return v * 0.5 * (1.0 + jax.lax.erf(v * _INV_SQRT2))


def _shift_rows(a, off):
    n, c = a.shape
    if off > 0:
        return jnp.concatenate(
            [a[off:], jnp.zeros((off, c), a.dtype)], axis=0)
    if off < 0:
        return jnp.concatenate(
            [jnp.zeros((-off, c), a.dtype), a[:off]], axis=0)
    return a


def _softmax(x):
    m = jnp.max(x, axis=1, keepdims=True)
    e = jnp.exp(x - m)
    return e / jnp.sum(e, axis=1, keepdims=True)


def _conv_stack(a, m_neg, m_pos, ones8):
    return jnp.concatenate(
        [m_neg * _shift_rows(a, -1), a, m_pos * _shift_rows(a, 1), ones8],
        axis=1)


def _encoder(x2b, m_neg, m_pos, ones8, a1, a2, q, fcb):
    r1 = jax.nn.relu(_dot(_conv_stack(x2b, m_neg, m_pos, ones8), a1))
    r1b = r1.astype(jnp.bfloat16)
    r2 = jax.nn.relu(_dot(_conv_stack(r1b, m_neg, m_pos, ones8), a2))
    t = _dot(r2.astype(jnp.bfloat16), q)
    return t.reshape(F, HW, C).sum(axis=1) + fcb


def _enc_body(*refs):
    raw_ref, norm_ref = refs[0], refs[1]
    a1s, a2s, qs, fcbs = [r[...] for r in refs[2:6]]
    feat_out, amp_out = refs[6], refs[7]

    raw = raw_ref[...]
    norm = norm_ref[...]
    delta = jnp.concatenate(
        [jnp.zeros_like(norm[:, :1]), norm[:, 1:] - norm[:, :-1]], axis=1)

    hrow = jax.lax.broadcasted_iota(jnp.int32, (R, 1), 0) % HW
    m_neg = (hrow >= 1).astype(jnp.bfloat16)
    m_pos = (hrow <= HW - 2).astype(jnp.bfloat16)
    ones8 = jnp.ones((R, 8), jnp.bfloat16)

    outs = []
    for k, x4 in enumerate((raw, norm, delta)):
        x2b = x4.reshape(R, HW)
        if x2b.dtype != jnp.bfloat16:
            x2b = x2b.astype(jnp.bfloat16)
        outs.append(_encoder(x2b, m_neg, m_pos, ones8,
                             a1s[k], a2s[k], qs[k], fcbs[k]))
    feat_out[...] = jnp.concatenate(outs, axis=1)
    amp_out[...] = raw.sum(axis=3).sum(axis=2) * np.float32(1.0 / (HW * HW))


def _seq_body(refs, n_params):
    feat_ref, amp_ref, tab_ref = refs[0], refs[1], refs[2]
    c = [r[...] for r in refs[3:3 + n_params]]
    ol, oo, org, oprob = refs[3 + n_params:]

    (ti_w, bn_sc, bn_b, wacc, bacc, gp_m, exp_m, tsum_m,
     tr_w, tb1_w, tb2_w, t2t_w, fus_w,
     sc1_w, sc2_w, so1_w, so2_w, sr1_w, sr2_w, biases) = c

    def bia(name):
        o = _BIAS_OFF[name]
        w = dict(_BIAS_PARTS)[name]
        return biases[:, o:o + w]

    gp_b, tr_b, ln_g, ln_b = bia('gp_b'), bia('tr_b'), bia('ln_g'), bia('ln_b')
    tb1_b, tb2_b, t2t_b = bia('tb1_b'), bia('tb2_b'), bia('t2t_b')
    fus_b, sc1_b, sc2_b = bia('fus_b'), bia('sc1_b'), bia('sc2_b')
    so1_b, so2_b, sr1_b, sr2_b = (bia('so1_b'), bia('so2_b'),
                                  bia('sr1_b'), bia('sr2_b'))

    feat = feat_ref[...]
    amp = amp_ref[...]
    tab = tab_ref[...]

    s = jax.nn.relu(_dot(feat, ti_w) * bn_sc + bn_b)
    s3 = s.reshape(BS2, SEQ, 48)
    st = jnp.concatenate([s3[:, t] for t in range(SEQ)], axis=1)

    for i in range(3):
        st = jax.nn.relu(st + _dot(st, wacc[i]) + bacc[i])

    scores = _dot(st, gp_m) + gp_b
    attn = _softmax(scores)

    thr = amp.mean(axis=1, keepdims=True)
    m_hi = (amp >= thr).astype(jnp.float32)
    m_lo = 1.0 - m_hi

    gfeat = _dot(st * _dot(attn, exp_m), tsum_m)
    ph = (_dot(st * _dot(m_hi, exp_m), tsum_m) /
          (jnp.sum(m_hi, axis=1, keepdims=True) + 1e-6))
    plo = (_dot(st * _dot(m_lo, exp_m), tsum_m) /
           (jnp.sum(m_lo, axis=1, keepdims=True) + 1e-6))

    fused = jnp.concatenate([gfeat, ph, plo], axis=1)
    trunk = jax.nn.relu(_dot(fused, tr_w) + tr_b)

    mu = tab.mean(axis=1, keepdims=True)
    xc = tab - mu
    var = (xc * xc).mean(axis=1, keepdims=True)
    tn = xc / jnp.sqrt(var + 1e-5) * ln_g + ln_b
    tf = _gelu(_dot(tn, tb1_w) + tb1_b)
    tf = _gelu(_dot(tf, tb2_w) + tb2_b)
    tp = _dot(tf, t2t_w) + t2t_b

    hyb = jax.nn.relu(
        _dot(jnp.concatenate([trunk, tp, trunk * tp], axis=1), fus_w) + fus_b)
    sl = _dot(jax.nn.relu(_dot(hyb, sc1_w) + sc1_b), sc2_w) + sc2_b
    so = _dot(jax.nn.relu(_dot(hyb, so1_w) + so1_b), so2_w) + so2_b
    sp = _softmax(sl)
    vals = (jax.lax.broadcasted_iota(jnp.int32, (1, NSZ), 1).astype(
        jnp.float32) * np.float32(1.0 / (NSZ - 1)))
    expected = jnp.sum(sp * vals, axis=1, keepdims=True)
    res = 0.35 * jnp.tanh(
        _dot(jax.nn.relu(_dot(hyb, sr1_w) + sr1_b), sr2_w) + sr2_b)
    rg = jnp.clip(expected + res, 0.0, 1.0)

    ol[...] = sl
    oo[...] = so
    org[...] = rg
    oprob[...] = sp


def _prep_enc_params(p):
    sel = jnp.asarray(_SEL)
    encs = ('amp', 'shp', 'dlt')
    w1 = jnp.stack([p[e + '_c1w'].reshape(3, 3, C) for e in encs])
    b1 = jnp.stack([p[e + '_c1b'] for e in encs])
    w2 = jnp.stack([p[e + '_c2w'] for e in encs])
    b2 = jnp.stack([p[e + '_c2b'] for e in encs])
    fcw = jnp.stack([p[e + '_fcw'] for e in encs])
    fcb = jnp.stack([p[e + '_fcb'] for e in encs])

    a1 = jnp.einsum('itw,eotc->eoiwc', sel, w1).reshape(3, 3 * HW, WC)
    b1t = jnp.tile(b1, (1, HW)).reshape(3, 1, WC)
    a1 = jnp.concatenate(
        [a1, b1t, jnp.zeros((3, 7, WC), jnp.float32)], axis=1)
    a2 = jnp.einsum('itw,eotcd->eoicwd', sel, w2).reshape(3, 3 * WC, WC)
    b2t = jnp.tile(b2, (1, HW)).reshape(3, 1, WC)
    a2 = jnp.concatenate(
        [a2, b2t, jnp.zeros((3, 7, WC), jnp.float32)], axis=1)
    q = jnp.tile(fcw * np.float32(1.0 / (HW * HW)), (1, HW, 1))
    return [a1.astype(jnp.bfloat16), a2.astype(jnp.bfloat16),
            q.astype(jnp.bfloat16), fcb.reshape(3, 1, C)]


_BIAS_PARTS = [('gp_b', 1), ('tr_b', 64), ('ln_g', NTAB), ('ln_b', NTAB),
               ('tb1_b', 64), ('tb2_b', 64), ('t2t_b', 64), ('fus_b', 96),
               ('sc1_b', 96), ('sc2_b', NSZ), ('so1_b', 48), ('so2_b', 6),
               ('sr1_b', 96), ('sr2_b', 1)]
_BIAS_OFF = {}
_off = 0
for _n, _w in _BIAS_PARTS:
    _BIAS_OFF[_n] = _off
    _off += _w
_BIAS_TOT = _off


def _prep_seq_params(p):
    dsel = jnp.asarray(np.stack([_DSEL[d] for d in (1, 2, 4)]))
    w_all = jnp.stack([
        jnp.stack([p[f'blk{i}_d{d}_w'] for d in (1, 2, 4)])
        for i in range(3)])
    b_all = jnp.stack([
        jnp.stack([p[f'blk{i}_d{d}_b'] for d in (1, 2, 4)])
        for i in range(3)])
    wacc = jnp.einsum('dutp,idpcm->iuctm', dsel,
                      w_all).reshape(3, TC48, TC48)
    bacc = jnp.tile(b_all.sum(axis=1), (1, SEQ)).reshape(3, 1, TC48)
    gp_m = jnp.kron(jnp.eye(SEQ, dtype=jnp.float32),
                    p['gp_w'].reshape(48, 1))
    biases = jnp.concatenate(
        [p[n].reshape(-1) for n, _ in _BIAS_PARTS]).reshape(1, _BIAS_TOT)
    out = [p['ti_w'].reshape(72, 48),
           p['bn_g'].reshape(1, 48) * np.float32(1.0 / np.sqrt(1.0 + 1e-5)),
           p['bn_b'].reshape(1, 48),
           wacc, bacc, gp_m,
           jnp.asarray(_EXP), jnp.asarray(_TSUM),
           p['tr_w'], p['tb1_w'], p['tb2_w'], p['t2t_w'], p['fus_w'],
           p['sc1_w'], p['sc2_w'], p['so1_w'], p['so2_w'], p['sr1_w'],
           p['sr2_w'], biases]
    return out


def _full_spec(a):
    nd = a.ndim
    return pl.BlockSpec(a.shape, lambda i, _nd=nd: (0,) * _nd)


def kernel(raw_window, norm_window, tabular_x, params):
    enc_consts = _prep_enc_params(params)
    seq_consts = _prep_seq_params(params)

    enc_fn = pl.pallas_call(
        _enc_body,
        grid=(B // BS,),
        in_specs=[
            pl.BlockSpec((BS, SEQ, HW, HW), lambda i: (i, 0, 0, 0)),
            pl.BlockSpec((BS, SEQ, HW, HW), lambda i: (i, 0, 0, 0)),
        ] + [_full_spec(a) for a in enc_consts],
        out_specs=[
            pl.BlockSpec((F, 72), lambda i: (i, 0)),
            pl.BlockSpec((BS, SEQ), lambda i: (i, 0)),
        ],
        out_shape=[
            jax.ShapeDtypeStruct((B * SEQ, 72), jnp.float32),
            jax.ShapeDtypeStruct((B, SEQ), jnp.float32),
        ],
    )
    feat, amp = enc_fn(raw_window, norm_window.astype(jnp.bfloat16),
                       *enc_consts)

    n_seq = len(seq_consts)
    seq_fn = pl.pallas_call(
        lambda *refs: _seq_body(refs, n_seq),
        grid=(B // BS2,),
        in_specs=[
            pl.BlockSpec((F2, 72), lambda i: (i, 0)),
            pl.BlockSpec((BS2, SEQ), lambda i: (i, 0)),
            pl.BlockSpec((BS2, NTAB), lambda i: (i, 0)),
        ] + [_full_spec(a) for a in seq_consts],
        out_specs=[
            pl.BlockSpec((BS2, NSZ), lambda i: (i, 0)),
            pl.BlockSpec((BS2, 6), lambda i: (i, 0)),
            pl.BlockSpec((BS2, 1), lambda i: (i, 0)),
            pl.BlockSpec((BS2, NSZ), lambda i: (i, 0)),
        ],
        out_shape=[
            jax.ShapeDtypeStruct((B, NSZ), jnp.float32),
            jax.ShapeDtypeStruct((B, 6), jnp.float32),
            jax.ShapeDtypeStruct((B, 1), jnp.float32),
            jax.ShapeDtypeStruct((B, NSZ), jnp.float32),
        ],
    )
    return tuple(seq_fn(feat, amp, tabular_x, *seq_consts))
